# Initial kernel scaffold; baseline (speedup 1.0000x reference)
import jax
import jax.numpy as jnp
from jax.experimental import pallas as pl


def kernel(node_fts, edge_fts, edges, W_node, W_edge, a_node, a_edge):
    raise NotImplementedError("write your pallas kernel here")


# speedup vs baseline: 8.3835x; 8.3835x over previous
import functools

import jax
import jax.numpy as jnp
from jax import lax
from jax.experimental import pallas as pl
from jax.experimental.pallas import tpu as pltpu
from jax.experimental.pallas import tpu_sc as plsc

N = 10000
E = 320000
NPAD = 10240
NROWS = NPAD // 16
ALPHA = 0.2
NW = 32
EPT = E // NW
CH = 2000
NV = CH // 16
NCH = EPT // CH
RP = 128
NQP = 79
EPTP = NQP * RP
F = 64

def _mesh():
    return plsc.VectorSubcoreMesh(core_axis_name="c", subcore_axis_name="s")


_SC_PARAMS = pltpu.CompilerParams(needs_layout_passes=False,
                                  use_tc_tiling_on_sc=False)


def _f32(shape):
    return jax.ShapeDtypeStruct(shape, jnp.float32)


def _i32(shape):
    return jax.ShapeDtypeStruct(shape, jnp.int32)


def _tc_pre_body(nf_ref, efn_ref, wn_ref, we_ref, av_ref,
                 hv_ref, ev_ref, s_ref):
    hv = jnp.dot(nf_ref[...], wn_ref[...], preferred_element_type=jnp.float32)
    hv_ref[...] = hv
    ev_ref[...] = jnp.dot(efn_ref[...], we_ref[...],
                          preferred_element_type=jnp.float32)
    s_ref[...] = jnp.dot(hv, av_ref[...], preferred_element_type=jnp.float32)


def _tc_t2_body(ef_ref, ae2_ref, t2_ref):
    t2_ref[...] = jnp.dot(ef_ref[...], ae2_ref[...],
                          preferred_element_type=jnp.float32)


def _sc_edge_body(src_h, dst_h, t2_h, s1_h, s2_h, t1_h, rowidx_h,
                  na_h, ea_h, nasum_h, easum_h, cnt_h,
                  s1_v, s2_v, t1_v, acc_na, acc_ea, acc_cnt,
                  src_c, dst_c, t2_c, na_c, ea_c, idxr,
                  sh_na, sh_ea, sh_cnt):
    c = lax.axis_index("c")
    s = lax.axis_index("s")
    wid = c * 16 + s
    zf = jnp.zeros((16,), jnp.float32)
    zi = jnp.zeros((16,), jnp.int32)
    ones = jnp.ones((16,), jnp.int32)

    def zrow(r, carry):
        acc_na[r] = zf
        acc_ea[r] = zf
        acc_cnt[r] = zi
        return carry

    lax.fori_loop(0, NROWS, zrow, 0)
    pltpu.sync_copy(rowidx_h, idxr)
    pltpu.sync_copy(s1_h, s1_v.at[pl.ds(0, N)])
    pltpu.sync_copy(s2_h, s2_v.at[pl.ds(0, N)])
    pltpu.sync_copy(t1_h, t1_v.at[pl.ds(0, N)])

    @pl.when(s == 0)
    def _zero_shared():
        pltpu.sync_copy(acc_na, sh_na)
        pltpu.sync_copy(acc_ea, sh_ea)
        pltpu.sync_copy(acc_cnt, sh_cnt)

    plsc.subcore_barrier()

    for ch in range(NCH):
        base = wid * EPT + ch * CH
        pltpu.sync_copy(src_h.at[pl.ds(base, CH)], src_c)
        pltpu.sync_copy(dst_h.at[pl.ds(base, CH)], dst_c)
        pltpu.sync_copy(t2_h.at[pl.ds(base, CH)], t2_c)

        def body(i, carry):
            o = pl.multiple_of(i * 16, 16)
            isv = src_c[pl.ds(o, 16)]
            idv = dst_c[pl.ds(o, 16)]
            sv = plsc.load_gather(s1_v, [isv]) + plsc.load_gather(s2_v, [idv])
            sv = jnp.maximum(sv, ALPHA * sv)
            nav = jnp.exp(jnp.clip(sv, -2.0, 2.0))
            tv = plsc.load_gather(t1_v, [isv]) + t2_c[pl.ds(o, 16)]
            tv = jnp.maximum(tv, ALPHA * tv)
            eav = jnp.exp(jnp.clip(tv, -2.0, 2.0))
            na_c[pl.ds(o, 16)] = nav
            ea_c[pl.ds(o, 16)] = eav
            rows = lax.shift_right_logical(isv, 4)
            cols = lax.bitwise_and(isv, 15)
            plsc.addupdate_scatter(acc_na, [rows, cols], nav)
            plsc.addupdate_scatter(acc_ea, [rows, cols], eav)
            plsc.addupdate_scatter(acc_cnt, [rows, cols], ones)
            return carry

        lax.fori_loop(0, NV, body, 0)
        pltpu.sync_copy(na_c, na_h.at[pl.ds(base, CH)])
        pltpu.sync_copy(ea_c, ea_h.at[pl.ds(base, CH)])

    for j in range(NROWS // 128):
        sl = pl.ds(128 * j, 128)
        pltpu.sync_copy(acc_na.at[sl], sh_na.at[idxr.at[j]], add=True)
        pltpu.sync_copy(acc_ea.at[sl], sh_ea.at[idxr.at[j]], add=True)
        pltpu.sync_copy(acc_cnt.at[sl], sh_cnt.at[idxr.at[j]], add=True)

    plsc.subcore_barrier()

    @pl.when(s == 0)
    def _write_out():
        pltpu.sync_copy(sh_na, nasum_h.at[c])
        pltpu.sync_copy(sh_ea, easum_h.at[c])
        pltpu.sync_copy(sh_cnt, cnt_h.at[c])


def _sc_merge_body(nsa_h, nsb_h, esa_h, esb_h, ca_h, cb_h,
                   nasum_h, easum_h, off_h,
                   c0_v, c1_v, off_v, buf_a, buf_b):
    c = lax.axis_index("c")
    s = lax.axis_index("s")
    wid = c * 16 + s

    @pl.when(wid == 0)
    def _scan_counts():
        pltpu.sync_copy(ca_h, c0_v)
        pltpu.sync_copy(cb_h, c1_v)

        def body(i, carry):
            o = pl.multiple_of(i * 16, 16)
            v = c0_v[pl.ds(o, 16)] + c1_v[pl.ds(o, 16)]
            cs = plsc.cumsum(v)
            off_v[pl.ds(o, 16)] = (carry + cs) - v
            return carry + cs[15]

        lax.fori_loop(0, NPAD // 16, body, jnp.int32(0))
        pltpu.sync_copy(off_v, off_h)

    CHK = NPAD // 8

    @pl.when((wid >= 1) & (wid <= 8))
    def _merge_na():
        o = pl.multiple_of((wid - 1) * CHK, 8)
        pltpu.sync_copy(nsa_h.at[pl.ds(o, CHK)], buf_a)
        pltpu.sync_copy(nsb_h.at[pl.ds(o, CHK)], buf_b)

        def body(i, carry):
            sl = pl.ds(pl.multiple_of(i * 16, 16), 16)
            buf_a[sl] = buf_a[sl] + buf_b[sl]
            return carry

        lax.fori_loop(0, CHK // 16, body, 0)
        pltpu.sync_copy(buf_a, nasum_h.at[pl.ds(o, CHK)])

    @pl.when((wid >= 9) & (wid <= 16))
    def _merge_ea():
        o = pl.multiple_of((wid - 9) * CHK, 8)
        pltpu.sync_copy(esa_h.at[pl.ds(o, CHK)], buf_a)
        pltpu.sync_copy(esb_h.at[pl.ds(o, CHK)], buf_b)

        def body(i, carry):
            sl = pl.ds(pl.multiple_of(i * 16, 16), 16)
            buf_a[sl] = buf_a[sl] + buf_b[sl]
            return carry

        lax.fori_loop(0, CHK // 16, body, 0)
        pltpu.sync_copy(buf_a, easum_h.at[pl.ds(o, CHK)])


def _sc_norm_body(na_h, ea_h, nasum_h, easum_h, off_h,
                  nn_h, ne_h, varp_h,
                  nasum_v, easum_v, off_v, arr,
                  na_t, ea_t, nn_t, ne_t, varrow):
    c = lax.axis_index("c")
    s = lax.axis_index("s")
    wid = c * 16 + s
    base = wid * EPT
    zf = jnp.zeros((16,), jnp.float32)
    zi = jnp.zeros((16,), jnp.int32)
    ones = jnp.ones((16,), jnp.int32)

    pltpu.sync_copy(nasum_h, nasum_v)
    pltpu.sync_copy(easum_h, easum_v)
    pltpu.sync_copy(off_h, off_v)
    pltpu.sync_copy(na_h.at[pl.ds(base, EPT)], na_t)
    pltpu.sync_copy(ea_h.at[pl.ds(base, EPT)], ea_t)

    def bs(i, lohi):
        lo, hi = lohi
        mid = lax.div(lo + hi + 1, 2)
        p = off_v[pl.ds(mid, 16)][0] <= base
        return (jnp.where(p, mid, lo), jnp.where(p, hi, mid - 1))

    j0, _ = lax.fori_loop(0, 14, bs, (jnp.int32(0), jnp.int32(N - 1)))

    def za(i, carry):
        arr[pl.ds(pl.multiple_of(i * 16, 16), 16)] = zi
        return carry

    lax.fori_loop(0, EPT // 16, za, 0)

    def hb(k, carry):
        ov = off_v[pl.ds(pl.multiple_of(k * 16, 16), 16)]
        m = (ov > base) & (ov < base + EPT)
        plsc.addupdate_scatter(arr, [ov - base], ones, mask=m)
        return carry

    lax.fori_loop(0, NPAD // 16, hb, 0)

    def nb(i, carry):
        gcar, sn, qn, se, qe = carry
        o = pl.multiple_of(i * 16, 16)
        hv16 = arr[pl.ds(o, 16)]
        cs = plsc.cumsum(hv16)
        gv = (j0 + gcar) + cs
        rn = plsc.load_gather(nasum_v, [gv])
        re = plsc.load_gather(easum_v, [gv])
        nn = na_t[pl.ds(o, 16)] / rn
        ne = ea_t[pl.ds(o, 16)] / re
        nn_t[pl.ds(o, 16)] = nn
        ne_t[pl.ds(o, 16)] = ne
        return (gcar + cs[15], sn + nn, qn + nn * nn, se + ne, qe + ne * ne)

    _, sn, qn, se, qe = lax.fori_loop(
        0, EPT // 16, nb, (jnp.int32(0), zf, zf, zf, zf))
    varrow[0] = sn
    varrow[1] = qn
    varrow[2] = se
    varrow[3] = qe
    pltpu.sync_copy(nn_t, nn_h.at[pl.ds(base, EPT)])
    pltpu.sync_copy(ne_t, ne_h.at[pl.ds(base, EPT)])
    pltpu.sync_copy(varrow, varp_h.at[wid])


def _sc_aggr_body(src2d_h, dst2d_h, w_h, tab_h, outp_h,
                  src2_t, dst2_t, w_t, grows, wrows, sh_acc):
    c = lax.axis_index("c")
    s = lax.axis_index("s")
    wid = c * 16 + s
    zf = jnp.zeros((16,), jnp.float32)

    pltpu.sync_copy(src2d_h.at[pl.ds(wid * NQP, NQP)], src2_t)
    pltpu.sync_copy(dst2d_h.at[pl.ds(wid * NQP, NQP)], dst2_t)
    pltpu.sync_copy(w_h.at[pl.ds(wid * EPTP, EPTP)], w_t.at[pl.ds(0, EPTP)])

    def zr(k, carry):
        for m in range(F // 16):
            wrows[0, k, pl.ds(16 * m, 16)] = zf
        return carry

    lax.fori_loop(0, RP, zr, 0)
    for q in range(5):
        ro = pl.multiple_of(s * (N // 16) + q * 125, 125)
        pltpu.sync_copy(wrows.at[0].at[pl.ds(0, 125)],
                        sh_acc.at[pl.ds(ro, 125)])
    plsc.subcore_barrier()

    def outer(q, carry):
        pltpu.sync_copy(tab_h.at[dst2_t.at[q]], grows.at[0])

        def rowm(k, carry2):
            w = w_t[pl.ds(q * RP + k, 16)][0]
            for m in range(F // 16):
                sl = pl.ds(16 * m, 16)
                wrows[0, k, sl] = grows[0, k, sl] * w
            return carry2

        lax.fori_loop(0, RP, rowm, 0)
        pltpu.sync_copy(wrows.at[0], sh_acc.at[src2_t.at[q]], add=True)
        return carry

    lax.fori_loop(0, NQP, outer, 0)

    plsc.subcore_barrier()

    @pl.when(s == 0)
    def _write_out():
        pltpu.sync_copy(sh_acc, outp_h.at[c])


def _tc_merge_body(np_ref, ep_ref, vp_ref, nout_ref, eout_ref,
                   nav_ref, eav_ref):
    nout_ref[...] = np_ref[0] + np_ref[1]
    eout_ref[...] = ep_ref[0] + ep_ref[1]
    vp = vp_ref[...]
    sn = jnp.sum(vp[:, 0, :])
    qn = jnp.sum(vp[:, 1, :])
    se = jnp.sum(vp[:, 2, :])
    qe = jnp.sum(vp[:, 3, :])
    denom = 1.0 / (E - 1)
    nav_ref[...] = jnp.reshape((qn - sn * sn / E) * denom, (1, 1))
    eav_ref[...] = jnp.reshape((qe - se * se / E) * denom, (1, 1))


def kernel_staged(node_fts, edge_fts, edges, W_node, W_edge, a_node, a_edge,
                  stage=6):
    node_fts = jnp.squeeze(node_fts)
    edge_fts = jnp.squeeze(edge_fts)
    edges = jnp.squeeze(edges)
    er = edges.reshape(E, 2)
    src = er[:, 0]
    dst = er[:, 1]
    pad2 = ((0, 0), (0, EPTP - EPT))
    src2d = jnp.pad(src.reshape(NW, EPT), pad2).reshape(NW * NQP, RP)
    dst2d = jnp.pad(dst.reshape(NW, EPT), pad2).reshape(NW * NQP, RP)
    rowidx = jnp.arange(NROWS, dtype=jnp.int32).reshape(NROWS // 128, 128)

    avecs = jnp.concatenate([a_node[:F], a_node[F:], a_edge[:F]], axis=1)
    hv, ev, svec = pl.pallas_call(
        _tc_pre_body,
        out_shape=[_f32((N, F)), _f32((N, F)), _f32((N, 4))],
    )(node_fts, edge_fts[:N], W_node, W_edge,
      jnp.pad(avecs, ((0, 0), (0, 1))))
    s1 = svec[:, 0]
    s2 = svec[:, 1]
    t1 = svec[:, 2]

    BE = 8000
    t2 = pl.pallas_call(
        _tc_t2_body,
        grid=(E // BE,),
        in_specs=[pl.BlockSpec((BE, 16), lambda i: (i, 0)),
                  pl.BlockSpec((16, 1), lambda i: (0, 0))],
        out_specs=pl.BlockSpec((BE, 1), lambda i: (i, 0)),
        out_shape=_f32((E, 1)),
    )(edge_fts, a_edge[F:F + 16])
    t2 = t2.reshape(E)
    if stage == 1:
        return (hv, ev, s1, s2, t1, t2)

    k2 = pl.kernel(
        _sc_edge_body,
        out_type=[_f32((E,)), _f32((E,)),
                  _f32((2, NROWS, 16)), _f32((2, NROWS, 16)),
                  _i32((2, NROWS, 16))],
        mesh=_mesh(),
        compiler_params=_SC_PARAMS,
        scratch_types=[
            pltpu.VMEM((NPAD,), jnp.float32), pltpu.VMEM((NPAD,), jnp.float32),
            pltpu.VMEM((NPAD,), jnp.float32),
            pltpu.VMEM((NROWS, 16), jnp.float32),
            pltpu.VMEM((NROWS, 16), jnp.float32),
            pltpu.VMEM((NROWS, 16), jnp.int32),
            pltpu.VMEM((CH,), jnp.int32), pltpu.VMEM((CH,), jnp.int32),
            pltpu.VMEM((CH,), jnp.float32), pltpu.VMEM((CH,), jnp.float32),
            pltpu.VMEM((CH,), jnp.float32),
            pltpu.VMEM((NROWS // 128, 128), jnp.int32),
            pltpu.VMEM_SHARED((NROWS, 16), jnp.float32),
            pltpu.VMEM_SHARED((NROWS, 16), jnp.float32),
            pltpu.VMEM_SHARED((NROWS, 16), jnp.int32),
        ],
    )
    na, ea, nasum_p, easum_p, cnt_p = k2(src, dst, t2, s1, s2, t1, rowidx)
    nasum_p = nasum_p.reshape(2, NPAD)
    easum_p = easum_p.reshape(2, NPAD)
    cnt_p = cnt_p.reshape(2, NPAD)
    if stage == 2:
        return (na, ea, nasum_p, easum_p, cnt_p)

    k3 = pl.kernel(
        _sc_merge_body,
        out_type=[_f32((NPAD,)), _f32((NPAD,)), _i32((NPAD,))],
        mesh=_mesh(),
        compiler_params=_SC_PARAMS,
        scratch_types=[
            pltpu.VMEM((NPAD,), jnp.int32), pltpu.VMEM((NPAD,), jnp.int32),
            pltpu.VMEM((NPAD,), jnp.int32),
            pltpu.VMEM((NPAD // 8,), jnp.float32),
            pltpu.VMEM((NPAD // 8,), jnp.float32),
        ],
    )
    nasum, easum, off = k3(nasum_p[0], nasum_p[1], easum_p[0], easum_p[1],
                           cnt_p[0], cnt_p[1])
    if stage == 3:
        return (nasum, easum, off)

    k4 = pl.kernel(
        _sc_norm_body,
        out_type=[_f32((E,)), _f32((E,)), _f32((NW, 4, 16))],
        mesh=_mesh(),
        compiler_params=_SC_PARAMS,
        scratch_types=[
            pltpu.VMEM((NPAD,), jnp.float32), pltpu.VMEM((NPAD,), jnp.float32),
            pltpu.VMEM((NPAD,), jnp.int32), pltpu.VMEM((EPT + 240,), jnp.int32),
            pltpu.VMEM((EPT,), jnp.float32), pltpu.VMEM((EPT,), jnp.float32),
            pltpu.VMEM((EPT,), jnp.float32), pltpu.VMEM((EPT,), jnp.float32),
            pltpu.VMEM((4, 16), jnp.float32),
        ],
    )
    nn, ne, varp = k4(na, ea, nasum, easum, off)
    if stage == 4:
        return (nn, ne, varp)

    k5 = pl.kernel(
        _sc_aggr_body,
        out_type=_f32((2, N, F)),
        mesh=_mesh(),
        compiler_params=_SC_PARAMS,
        scratch_types=[
            pltpu.VMEM((NQP, RP), jnp.int32), pltpu.VMEM((NQP, RP), jnp.int32),
            pltpu.VMEM((EPTP + 16,), jnp.float32),
            pltpu.VMEM((2, RP, F), jnp.float32),
            pltpu.VMEM((2, RP, F), jnp.float32),
            pltpu.VMEM_SHARED((N, F), jnp.float32),
        ],
    )
    nnp = jnp.pad(nn.reshape(NW, EPT), pad2).reshape(NW * EPTP)
    nep = jnp.pad(ne.reshape(NW, EPT), pad2).reshape(NW * EPTP)
    noutp = k5(src2d, dst2d, nnp, hv)
    eoutp = k5(src2d, dst2d, nep, ev)
    if stage == 5:
        return (noutp, eoutp, varp)

    node_out, edge_out, nav, eav = pl.pallas_call(
        _tc_merge_body,
        out_shape=[_f32((N, F)), _f32((N, F)), _f32((1, 1)), _f32((1, 1))],
    )(noutp, eoutp, varp)
    return (node_out, edge_out, nav.reshape(()), eav.reshape(()))


def kernel(node_fts, edge_fts, edges, W_node, W_edge, a_node, a_edge):
    return kernel_staged(node_fts, edge_fts, edges, W_node, W_edge,
                         a_node, a_edge, stage=6)
